# baseline (speedup 1.0000x reference)
import jax
import jax.numpy as jnp
from jax import lax
from jax.experimental import pallas as pl
from jax.experimental.pallas import tpu as pltpu
from jax.experimental.pallas import tpu_sc as plsc

_NUM_CODES = 1024
_EMB = 64
_S_TILE = 4096
_BLK = 8

_NC = 2
_NSUB = 16
_ROWS_PER_W = 8
_CHUNK = 2048
_LANES = 16


def _vq_tile_kernel(z_ref, tab_ref, tsq_ref, idx_ref, loss_ref):
    z = z_ref[0]
    table = tab_ref[...]
    s = z.shape[1]

    m2 = jax.lax.dot_general(
        table + table, z, (((1,), (0,)), ((), ())),
        preferred_element_type=jnp.float32,
    )

    t_sq = tsq_ref[...]
    z_sq = jnp.sum(z * z, axis=0)
    z_sq_row = z_sq[None, :]

    n_blocks = _NUM_CODES // _BLK
    minval = None
    minblk = None
    for k in range(n_blocks):
        a = z_sq_row + t_sq[k * _BLK:(k + 1) * _BLK, :]
        d_blk = a - m2[k * _BLK:(k + 1) * _BLK, :]
        if k == 0:
            minval = d_blk
            minblk = jnp.zeros((_BLK, s), jnp.int32)
        else:
            lt = d_blk < minval
            minval = jnp.where(lt, d_blk, minval)
            minblk = jnp.where(lt, k, minblk)

    dmin = jnp.min(minval, axis=0)
    sub_iota = jax.lax.broadcasted_iota(jnp.int32, (_BLK, s), 0)
    code = minblk * _BLK + sub_iota
    cand = jnp.where(minval == dmin[None, :], code, _NUM_CODES)
    idx = jnp.min(cand, axis=0)

    idx_ref[0, 0, 0] = idx
    loss_ref[0, 0, 0, 0] = jnp.sum(dmin)


def _sc_lookup_kernel(tabt_hbm, idx_hbm, out_hbm, tabv, idxv, outv):
    wid = lax.axis_index("s") * _NC + lax.axis_index("c")
    nw = _NC * _NSUB
    tiles_per_b = nw // 4
    b = wid // tiles_per_b
    e_base = (wid % tiles_per_b) * _ROWS_PER_W

    pltpu.sync_copy(
        tabt_hbm.at[pl.ds(e_base * _NUM_CODES, _ROWS_PER_W * _NUM_CODES)],
        tabv)

    spatial = 16384
    n_chunks = spatial // _CHUNK
    for chunk in range(n_chunks):
        pltpu.sync_copy(
            idx_hbm.at[pl.ds(b * spatial + chunk * _CHUNK, _CHUNK)], idxv)

        def body(i, carry):
            vidx = idxv[pl.ds(i * _LANES, _LANES)]
            for e in range(_ROWS_PER_W):
                g = plsc.load_gather(tabv, [vidx + e * _NUM_CODES])
                outv[e, pl.ds(i * _LANES, _LANES)] = g
            return carry

        lax.fori_loop(0, _CHUNK // _LANES, body, 0, unroll=4)

        pltpu.sync_copy(
            outv,
            out_hbm.at[pl.ds(b * _EMB + e_base, _ROWS_PER_W),
                       pl.ds(chunk * _CHUNK, _CHUNK)])


def kernel(z, table):
    b, emb, d_, h, w = z.shape
    spatial = d_ * h * w
    z3 = z.reshape(b, emb, spatial)
    ns = spatial // _S_TILE

    idx4, loss4 = pl.pallas_call(
        _vq_tile_kernel,
        grid=(b, ns),
        in_specs=[
            pl.BlockSpec((1, emb, _S_TILE), lambda i, j: (i, 0, j)),
            pl.BlockSpec((_NUM_CODES, emb), lambda i, j: (0, 0)),
            pl.BlockSpec((_NUM_CODES, 1), lambda i, j: (0, 0)),
        ],
        out_specs=[
            pl.BlockSpec((1, 1, 1, _S_TILE), lambda i, j: (i, j, 0, 0)),
            pl.BlockSpec((1, 1, 1, 1), lambda i, j: (i, j, 0, 0),
                         memory_space=pltpu.SMEM),
        ],
        out_shape=[
            jax.ShapeDtypeStruct((b, ns, 1, _S_TILE), jnp.int32),
            jax.ShapeDtypeStruct((b, ns, 1, 1), jnp.float32),
        ],
    )(z3, table, jnp.sum(table ** 2, axis=1)[:, None])

    indices = idx4.reshape(b * spatial)

    tabt = table.T.reshape(-1)
    mesh = plsc.VectorSubcoreMesh(core_axis_name="c", subcore_axis_name="s")
    zq2 = pl.kernel(
        _sc_lookup_kernel,
        out_type=jax.ShapeDtypeStruct((b * emb, spatial), jnp.float32),
        mesh=mesh,
        scratch_types=[
            pltpu.VMEM((_ROWS_PER_W * _NUM_CODES,), jnp.float32),
            pltpu.VMEM((_CHUNK,), jnp.int32),
            pltpu.VMEM((_ROWS_PER_W, _CHUNK), jnp.float32),
        ],
        compiler_params=pltpu.CompilerParams(needs_layout_passes=False),
    )(tabt, indices)

    z_q_st = zq2.reshape(b, emb, d_, h, w)
    n_elems = b * emb * spatial
    vq_loss = jnp.sum(loss4) * jnp.float32(1.25 / n_elems)
    return (z_q_st, vq_loss, indices)


# speedup vs baseline: 1.0550x; 1.0550x over previous
import jax
import jax.numpy as jnp
from jax import lax
from jax.experimental import pallas as pl
from jax.experimental.pallas import tpu as pltpu
from jax.experimental.pallas import tpu_sc as plsc

_NUM_CODES = 1024
_EMB = 64
_S_TILE = 4096
_BLK = 8

_NC = 2
_NSUB = 16
_ROWS_PER_W = 8
_CHUNK = 2048
_LANES = 16


def _vq_tile_kernel(z_ref, tab_ref, tsq_ref, idx_ref, loss_ref):
    z = z_ref[0]
    table = tab_ref[...]
    s = z.shape[1]

    m2 = jax.lax.dot_general(
        table + table, z, (((1,), (0,)), ((), ())),
        preferred_element_type=jnp.float32,
    )

    t_sq = tsq_ref[...]
    z_sq = jnp.sum(z * z, axis=0)
    z_sq_row = z_sq[None, :]

    n_blocks = _NUM_CODES // _BLK
    minval = None
    minblk = None
    for k in range(n_blocks):
        a = z_sq_row + t_sq[k * _BLK:(k + 1) * _BLK, :]
        d_blk = a - m2[k * _BLK:(k + 1) * _BLK, :]
        if k == 0:
            minval = d_blk
            minblk = jnp.zeros((_BLK, s), jnp.int32)
        else:
            lt = d_blk < minval
            minval = jnp.where(lt, d_blk, minval)
            minblk = jnp.where(lt, k, minblk)

    dmin = jnp.min(minval, axis=0)
    sub_iota = jax.lax.broadcasted_iota(jnp.int32, (_BLK, s), 0)
    code = minblk * _BLK + sub_iota
    cand = jnp.where(minval == dmin[None, :], code, _NUM_CODES)
    idx = jnp.min(cand, axis=0)

    idx_ref[0, 0, 0] = idx
    loss_ref[0, 0, 0, 0] = jnp.sum(dmin)


def _sc_lookup_kernel(tabt_hbm, idx_hbm, out_hbm, tabv, idxv, outv,
                      si0, si1, so0, so1):
    wid = lax.axis_index("s") * _NC + lax.axis_index("c")
    nw = _NC * _NSUB
    tiles_per_b = nw // 4
    b = wid // tiles_per_b
    e_base = (wid % tiles_per_b) * _ROWS_PER_W

    pltpu.sync_copy(
        tabt_hbm.at[pl.ds(e_base * _NUM_CODES, _ROWS_PER_W * _NUM_CODES)],
        tabv)

    spatial = 16384
    n_chunks = spatial // _CHUNK
    sems_i = (si0, si1)
    sems_o = (so0, so1)

    def idx_copy(c, p):
        return pltpu.async_copy(
            idx_hbm.at[pl.ds(b * spatial + c * _CHUNK, _CHUNK)],
            idxv.at[p], sems_i[p])

    def out_copy(c, p):
        return pltpu.async_copy(
            outv.at[p],
            out_hbm.at[pl.ds(b * _EMB + e_base, _ROWS_PER_W),
                       pl.ds(c * _CHUNK, _CHUNK)],
            sems_o[p])

    pending_out = [None, None]
    idx_pending = idx_copy(0, 0)
    for chunk in range(n_chunks):
        p = chunk & 1
        idx_pending.wait()
        if chunk + 1 < n_chunks:
            idx_next = idx_copy(chunk + 1, 1 - p)
        if pending_out[p] is not None:
            pending_out[p].wait()

        def body(i, carry):
            vidx = idxv[p, pl.ds(i * _LANES, _LANES)]
            for e in range(_ROWS_PER_W):
                g = plsc.load_gather(tabv, [vidx + e * _NUM_CODES])
                outv[p, e, pl.ds(i * _LANES, _LANES)] = g
            return carry

        lax.fori_loop(0, _CHUNK // _LANES, body, 0, unroll=8)

        pending_out[p] = out_copy(chunk, p)
        if chunk + 1 < n_chunks:
            idx_pending = idx_next
    pending_out[0].wait()
    pending_out[1].wait()


def kernel(z, table):
    b, emb, d_, h, w = z.shape
    spatial = d_ * h * w
    z3 = z.reshape(b, emb, spatial)
    ns = spatial // _S_TILE

    idx4, loss4 = pl.pallas_call(
        _vq_tile_kernel,
        grid=(b, ns),
        in_specs=[
            pl.BlockSpec((1, emb, _S_TILE), lambda i, j: (i, 0, j)),
            pl.BlockSpec((_NUM_CODES, emb), lambda i, j: (0, 0)),
            pl.BlockSpec((_NUM_CODES, 1), lambda i, j: (0, 0)),
        ],
        out_specs=[
            pl.BlockSpec((1, 1, 1, _S_TILE), lambda i, j: (i, j, 0, 0)),
            pl.BlockSpec((1, 1, 1, 1), lambda i, j: (i, j, 0, 0),
                         memory_space=pltpu.SMEM),
        ],
        out_shape=[
            jax.ShapeDtypeStruct((b, ns, 1, _S_TILE), jnp.int32),
            jax.ShapeDtypeStruct((b, ns, 1, 1), jnp.float32),
        ],
    )(z3, table, jnp.sum(table ** 2, axis=1)[:, None])

    indices = idx4.reshape(b * spatial)

    tabt = table.T.reshape(-1)
    mesh = plsc.VectorSubcoreMesh(core_axis_name="c", subcore_axis_name="s")
    zq2 = pl.kernel(
        _sc_lookup_kernel,
        out_type=jax.ShapeDtypeStruct((b * emb, spatial), jnp.float32),
        mesh=mesh,
        scratch_types=[
            pltpu.VMEM((_ROWS_PER_W * _NUM_CODES,), jnp.float32),
            pltpu.VMEM((2, _CHUNK), jnp.int32),
            pltpu.VMEM((2, _ROWS_PER_W, _CHUNK), jnp.float32),
            pltpu.SemaphoreType.DMA,
            pltpu.SemaphoreType.DMA,
            pltpu.SemaphoreType.DMA,
            pltpu.SemaphoreType.DMA,
        ],
        compiler_params=pltpu.CompilerParams(needs_layout_passes=False),
    )(tabt, indices)

    z_q_st = zq2.reshape(b, emb, d_, h, w)
    n_elems = b * emb * spatial
    vq_loss = jnp.sum(loss4) * jnp.float32(1.25 / n_elems)
    return (z_q_st, vq_loss, indices)


# speedup vs baseline: 1.6337x; 1.5485x over previous
import jax
import jax.numpy as jnp
from jax.experimental import pallas as pl
from jax.experimental.pallas import tpu as pltpu

_NUM_CODES = 1024
_EMB = 64
_S_TILE = 4096
_BLK = 8


def _vq_tile_kernel(z_ref, tab_ref, tsq_ref, zq_ref, idx_ref, loss_ref):
    z = z_ref[0]
    table = tab_ref[...]
    s = z.shape[1]

    m2 = jax.lax.dot_general(
        table + table, z, (((1,), (0,)), ((), ())),
        preferred_element_type=jnp.float32,
    )

    t_sq = tsq_ref[...]
    z_sq = jnp.sum(z * z, axis=0)
    z_sq_row = z_sq[None, :]

    n_blocks = _NUM_CODES // _BLK
    minval = None
    minblk = None
    for k in range(n_blocks):
        a = z_sq_row + t_sq[k * _BLK:(k + 1) * _BLK, :]
        d_blk = a - m2[k * _BLK:(k + 1) * _BLK, :]
        if k == 0:
            minval = d_blk
            minblk = jnp.zeros((_BLK, s), jnp.int32)
        else:
            lt = d_blk < minval
            minval = jnp.where(lt, d_blk, minval)
            minblk = jnp.where(lt, k, minblk)

    dmin = jnp.min(minval, axis=0)
    sub_iota = jax.lax.broadcasted_iota(jnp.int32, (_BLK, s), 0)
    code = minblk * _BLK + sub_iota
    cand = jnp.where(minval == dmin[None, :], code, _NUM_CODES)
    idx = jnp.min(cand, axis=0)

    iota = jax.lax.broadcasted_iota(jnp.int32, (_NUM_CODES, s), 0)
    oh = (iota == idx[None, :]).astype(jnp.float32)
    zq = jax.lax.dot_general(
        table, oh, (((0,), (0,)), ((), ())),
        preferred_element_type=jnp.float32,
    )

    zq_ref[0] = zq
    idx_ref[0, 0, 0] = idx
    loss_ref[0, 0, 0, 0] = jnp.sum(dmin)


def kernel(z, table):
    b, emb, d_, h, w = z.shape
    spatial = d_ * h * w
    z3 = z.reshape(b, emb, spatial)
    ns = spatial // _S_TILE

    zq3, idx4, loss4 = pl.pallas_call(
        _vq_tile_kernel,
        grid=(b, ns),
        in_specs=[
            pl.BlockSpec((1, emb, _S_TILE), lambda i, j: (i, 0, j)),
            pl.BlockSpec((_NUM_CODES, emb), lambda i, j: (0, 0)),
            pl.BlockSpec((_NUM_CODES, 1), lambda i, j: (0, 0)),
        ],
        out_specs=[
            pl.BlockSpec((1, emb, _S_TILE), lambda i, j: (i, 0, j)),
            pl.BlockSpec((1, 1, 1, _S_TILE), lambda i, j: (i, j, 0, 0)),
            pl.BlockSpec((1, 1, 1, 1), lambda i, j: (i, j, 0, 0),
                         memory_space=pltpu.SMEM),
        ],
        out_shape=[
            jax.ShapeDtypeStruct((b, emb, spatial), jnp.float32),
            jax.ShapeDtypeStruct((b, ns, 1, _S_TILE), jnp.int32),
            jax.ShapeDtypeStruct((b, ns, 1, 1), jnp.float32),
        ],
    )(z3, table, jnp.sum(table ** 2, axis=1)[:, None])

    z_q_st = zq3.reshape(b, emb, d_, h, w)
    indices = idx4.reshape(b * spatial)
    n_elems = b * emb * spatial
    vq_loss = jnp.sum(loss4) * jnp.float32(1.25 / n_elems)
    return (z_q_st, vq_loss, indices)
